# Initial kernel scaffold; baseline (speedup 1.0000x reference)
import jax
import jax.numpy as jnp
from jax.experimental import pallas as pl


def kernel(x, edge_index, W1, b1, W2, b2):
    raise NotImplementedError("write your pallas kernel here")


# speedup vs baseline: 6.8247x; 6.8247x over previous
import functools

import jax
import jax.numpy as jnp
from jax import lax
from jax.experimental import pallas as pl
from jax.experimental.pallas import tpu as pltpu
from jax.experimental.pallas import tpu_sc as plsc

N = 10000
NPAD = 10240
E = 160000
EPAD = 163840
ECH = EPAD // 128
F_IN = 1433
F_PAD = 1536
HID = 256
C = 7
CP = 16
K = 3

_MESH = dict(core_axis_name="c", subcore_axis_name="s")


def _deg_kernel(dst_hbm, out_hbm, acc, dstv, ones01, zbuf, sem):
    c = lax.axis_index("c")
    s = lax.axis_index("s")

    def zrow(i, _):
        for f in range(8):
            zbuf[i, pl.ds(f * 16, 16)] = jnp.zeros((16,), jnp.float32)
        return 0
    lax.fori_loop(0, 64, zrow, 0)

    def orow(i, _):
        ones01[i, pl.ds(0, 16)] = jnp.where(
            lax.iota(jnp.int32, 16) == 0, 1.0, 0.0)
        for f in range(1, 8):
            ones01[i, pl.ds(f * 16, 16)] = jnp.zeros((16,), jnp.float32)
        return 0
    lax.fori_loop(0, 128, orow, 0)

    for b in range(10):
        pltpu.sync_copy(zbuf, acc.at[pl.ds(s * 640 + b * 64, 64)])
    plsc.subcore_barrier()

    pltpu.sync_copy(dst_hbm.at[pl.ds(s * 80, 80)], dstv)

    def ebody(j, _):
        pltpu.sync_copy(ones01, acc.at[dstv.at[j]], add=True)
        return 0
    lax.fori_loop(0, 80, ebody, 0)
    plsc.subcore_barrier()

    r0 = c * 5120 + s * 320
    pltpu.sync_copy(acc.at[pl.ds(r0, 320)], out_hbm.at[pl.ds(r0, 320)])


def _degree(dst2d):
    k = pl.kernel(
        _deg_kernel,
        out_type=jax.ShapeDtypeStruct((NPAD, 128), jnp.float32),
        mesh=plsc.VectorSubcoreMesh(**_MESH),
        scratch_types=[
            pltpu.VMEM_SHARED((NPAD, 128), jnp.float32),
            pltpu.VMEM((80, 128), jnp.int32),
            pltpu.VMEM((128, 128), jnp.float32),
            pltpu.VMEM((64, 128), jnp.float32),
            pltpu.SemaphoreType.DMA,
        ],
    )
    return k(dst2d)


def _scales_body(deg_ref, dis_ref, dis2_ref):
    d = deg_ref[:, 0:1]
    pos = d > 0.0
    dsafe = jnp.maximum(d, 1e-12)
    ones = jnp.ones((1, 16), jnp.float32)
    dis_ref[...] = jnp.where(pos, lax.rsqrt(dsafe), 0.0) * ones
    dis2_ref[...] = jnp.where(pos, 1.0 / dsafe, 0.0) * ones


def _scales(deg16):
    return pl.pallas_call(
        _scales_body,
        grid=(10,),
        in_specs=[pl.BlockSpec((1024, 128), lambda i: (i, 0))],
        out_specs=[
            pl.BlockSpec((1024, 16), lambda i: (i, 0)),
            pl.BlockSpec((1024, 16), lambda i: (i, 0)),
        ],
        out_shape=[
            jax.ShapeDtypeStruct((NPAD, 16), jnp.float32),
            jax.ShapeDtypeStruct((NPAD, 16), jnp.float32),
        ],
    )(deg16)


def _m1_body(x_ref, w_ref, dis_ref, y_ref, g3_ref):
    j = pl.program_id(1)
    acc = jnp.dot(x_ref[...], w_ref[...], preferred_element_type=jnp.float32)
    ys = jnp.where(j >= 2, acc * dis_ref[:, 0:1], acc)
    y_ref[0, 0] = ys
    g3_ref[0] = ys


def _matmul1(x_pad, w1r, dis16):
    return pl.pallas_call(
        _m1_body,
        grid=(20, 8),
        in_specs=[
            pl.BlockSpec((512, F_PAD), lambda i, j: (i, 0)),
            pl.BlockSpec((F_PAD, 128), lambda i, j: (0, j)),
            pl.BlockSpec((512, 16), lambda i, j: (i, 0)),
        ],
        out_specs=[
            pl.BlockSpec((1, 1, 512, 128), lambda i, j: (j // 2, j % 2, i, 0)),
            pl.BlockSpec((1, 512, 128), lambda i, j: (j % 2, i, 0)),
        ],
        out_shape=[
            jax.ShapeDtypeStruct((K + 1, 2, NPAD, 128), jnp.float32),
            jax.ShapeDtypeStruct((2, NPAD, 128), jnp.float32),
        ],
    )(x_pad, w1r, dis16)


def _s1_kernel(g_hbm, src_hbm, dst_hbm, out_hbm, acc, srcv, dstv, rows,
               zbuf, sem):
    c = lax.axis_index("c")
    s = lax.axis_index("s")

    def zrow(i, _):
        for f in range(8):
            zbuf[i, pl.ds(f * 16, 16)] = jnp.zeros((16,), jnp.float32)
        return 0
    lax.fori_loop(0, 64, zrow, 0)
    for b in range(10):
        pltpu.sync_copy(zbuf, acc.at[pl.ds(s * 640 + b * 64, 64)])
    plsc.subcore_barrier()

    pltpu.sync_copy(src_hbm.at[pl.ds(s * 80, 80)], srcv)
    pltpu.sync_copy(dst_hbm.at[pl.ds(s * 80, 80)], dstv)

    def ebody(j, _):
        pltpu.async_copy(g_hbm.at[c].at[srcv.at[j]], rows, sem).wait()
        pltpu.sync_copy(rows, acc.at[dstv.at[j]], add=True)
        return 0
    lax.fori_loop(0, 80, ebody, 0)
    plsc.subcore_barrier()

    r0 = s * 640
    pltpu.sync_copy(acc.at[pl.ds(r0, 640)], out_hbm.at[c].at[pl.ds(r0, 640)])


def _agg1(g, src2d, dst2d):
    k = pl.kernel(
        _s1_kernel,
        out_type=jax.ShapeDtypeStruct((2, NPAD, 128), jnp.float32),
        mesh=plsc.VectorSubcoreMesh(**_MESH),
        scratch_types=[
            pltpu.VMEM_SHARED((NPAD, 128), jnp.float32),
            pltpu.VMEM((80, 128), jnp.int32),
            pltpu.VMEM((80, 128), jnp.int32),
            pltpu.VMEM((128, 128), jnp.float32),
            pltpu.VMEM((64, 128), jnp.float32),
            pltpu.SemaphoreType.DMA,
        ],
    )
    return k(g, src2d, dst2d)


def _c1_body(kk, yh_ref, s_ref, d2_ref, out_ref):
    del kk
    out_ref[0] = yh_ref[0, 0] + d2_ref[:, 0:1] * s_ref[0]


def _combine1(kk, y4, s, dis2_16):
    return pl.pallas_call(
        functools.partial(_c1_body, kk),
        grid=(2, 20),
        in_specs=[
            pl.BlockSpec((1, 1, 512, 128), lambda c, i: (kk, c, i, 0)),
            pl.BlockSpec((1, 512, 128), lambda c, i: (c, i, 0)),
            pl.BlockSpec((512, 16), lambda c, i: (i, 0)),
        ],
        out_specs=pl.BlockSpec((1, 512, 128), lambda c, i: (c, i, 0)),
        out_shape=jax.ShapeDtypeStruct((2, NPAD, 128), jnp.float32),
    )(y4, s, dis2_16)


def _m2_body(s_ref, y4_ref, dis_ref, b1_ref, w_ref, y2_ref, g2_ref):
    c = pl.program_id(1)
    h = jnp.maximum(
        y4_ref[0, 0] + dis_ref[:, 0:1] * s_ref[0] + b1_ref[0], 0.0)

    @pl.when(c == 0)
    def _():
        y2_ref[...] = jnp.zeros_like(y2_ref)

    y2_ref[...] += jnp.dot(h, w_ref[0], preferred_element_type=jnp.float32)

    @pl.when(c == 1)
    def _():
        col = lax.broadcasted_iota(jnp.int32, (512, (K + 1) * CP), 1)
        y2s = jnp.where(col < CP, y2_ref[...], y2_ref[...] * dis_ref[:, 0:1])
        y2_ref[...] = y2s
        g2_ref[...] = jnp.concatenate(
            [y2s[:, K * CP:(K + 1) * CP],
             jnp.zeros((512, 128 - CP), jnp.float32)], axis=1)


def _matmul2(s, y4, dis16, b1r, w2p):
    return pl.pallas_call(
        _m2_body,
        grid=(20, 2),
        in_specs=[
            pl.BlockSpec((1, 512, 128), lambda i, c: (c, i, 0)),
            pl.BlockSpec((1, 1, 512, 128), lambda i, c: (0, c, i, 0)),
            pl.BlockSpec((512, 16), lambda i, c: (i, 0)),
            pl.BlockSpec((1, 1, 128), lambda i, c: (c, 0, 0)),
            pl.BlockSpec((1, 128, (K + 1) * CP), lambda i, c: (c, 0, 0)),
        ],
        out_specs=[
            pl.BlockSpec((512, (K + 1) * CP), lambda i, c: (i, 0)),
            pl.BlockSpec((512, 128), lambda i, c: (i, 0)),
        ],
        out_shape=[
            jax.ShapeDtypeStruct((NPAD, (K + 1) * CP), jnp.float32),
            jax.ShapeDtypeStruct((NPAD, 128), jnp.float32),
        ],
    )(s, y4, dis16, b1r, w2p)


def _s2_kernel(g_hbm, src_hbm, dst_hbm, out_hbm, acc, srcv, dstv, rows,
               zbuf, sem):
    c = lax.axis_index("c")
    s = lax.axis_index("s")

    def zrow(i, _):
        for f in range(8):
            zbuf[i, pl.ds(f * 16, 16)] = jnp.zeros((16,), jnp.float32)
        return 0
    lax.fori_loop(0, 64, zrow, 0)
    for b in range(10):
        pltpu.sync_copy(zbuf, acc.at[pl.ds(s * 640 + b * 64, 64)])
    plsc.subcore_barrier()

    pltpu.sync_copy(src_hbm.at[pl.ds(s * 80, 80)], srcv)
    pltpu.sync_copy(dst_hbm.at[pl.ds(s * 80, 80)], dstv)

    def ebody(j, _):
        pltpu.async_copy(g_hbm.at[srcv.at[j]], rows, sem).wait()
        pltpu.sync_copy(rows, acc.at[dstv.at[j]], add=True)
        return 0
    lax.fori_loop(0, 80, ebody, 0)
    plsc.subcore_barrier()

    r0 = c * 5120 + s * 320
    pltpu.sync_copy(acc.at[pl.ds(r0, 320)], out_hbm.at[pl.ds(r0, 320)])


def _agg2(g, src2d, dst2d):
    k = pl.kernel(
        _s2_kernel,
        out_type=jax.ShapeDtypeStruct((NPAD, 128), jnp.float32),
        mesh=plsc.VectorSubcoreMesh(**_MESH),
        scratch_types=[
            pltpu.VMEM_SHARED((NPAD, 128), jnp.float32),
            pltpu.VMEM((80, 128), jnp.int32),
            pltpu.VMEM((80, 128), jnp.int32),
            pltpu.VMEM((128, 128), jnp.float32),
            pltpu.VMEM((64, 128), jnp.float32),
            pltpu.SemaphoreType.DMA,
        ],
    )
    return k(g, src2d, dst2d)


def _c2_body(yh_ref, s_ref, a2_ref, out_ref):
    v = yh_ref[0] + a2_ref[:, 0:1] * s_ref[:, 0:CP]
    out_ref[...] = jnp.concatenate(
        [v, jnp.zeros((1024, 128 - CP), jnp.float32)], axis=1)


def _combine2(kk, y2k, s, a2_16):
    return pl.pallas_call(
        _c2_body,
        grid=(10,),
        in_specs=[
            pl.BlockSpec((1, 1024, CP), lambda i: (kk, i, 0)),
            pl.BlockSpec((1024, 128), lambda i: (i, 0)),
            pl.BlockSpec((1024, 16), lambda i: (i, 0)),
        ],
        out_specs=pl.BlockSpec((1024, 128), lambda i: (i, 0)),
        out_shape=jax.ShapeDtypeStruct((NPAD, 128), jnp.float32),
    )(y2k, s, a2_16)


@jax.jit
def _run(x, edge_index, W1, b1, W2, b2):
    src = edge_index[0]
    dst = edge_index[1]
    src2d = jnp.concatenate(
        [src, jnp.zeros((EPAD - E,), jnp.int32)]).reshape(ECH, 128)
    dst2d = jnp.concatenate(
        [dst, jnp.full((EPAD - E,), N, jnp.int32)]).reshape(ECH, 128)

    x_pad = jnp.zeros((NPAD, F_PAD), jnp.float32).at[:N, :F_IN].set(x)
    w1r = jnp.zeros((F_PAD, (K + 1) * HID), jnp.float32).at[:F_IN].set(
        jnp.transpose(W1, (1, 0, 2)).reshape(F_IN, (K + 1) * HID))
    w2p = jnp.zeros((HID, K + 1, CP), jnp.float32).at[:, :, :C].set(
        jnp.transpose(W2, (1, 0, 2))).reshape(2, 128, (K + 1) * CP)
    b1r = b1.reshape(2, 1, 128)

    deg16 = _degree(dst2d)
    dis16, dis2_16 = _scales(deg16)

    y4, g = _matmul1(x_pad, w1r, dis16)
    g = _combine1(2, y4, _agg1(g, src2d, dst2d), dis2_16)
    g = _combine1(1, y4, _agg1(g, src2d, dst2d), dis2_16)
    sl = _agg1(g, src2d, dst2d)

    y2, g2 = _matmul2(sl, y4, dis16, b1r, w2p)
    y2k = jnp.transpose(y2.reshape(NPAD, K + 1, CP), (1, 0, 2))
    g2 = _combine2(2, y2k, _agg2(g2, src2d, dst2d), dis2_16)
    g2 = _combine2(1, y2k, _agg2(g2, src2d, dst2d), dis2_16)
    out = _combine2(0, y2k, _agg2(g2, src2d, dst2d), dis16)

    return out[:N, :C] + b2


def kernel(x, edge_index, W1, b1, W2, b2):
    return _run(x, edge_index, W1, b1, W2, b2)


# speedup vs baseline: 9.1775x; 1.3448x over previous
import functools

import jax
import jax.numpy as jnp
from jax import lax
from jax.experimental import pallas as pl
from jax.experimental.pallas import tpu as pltpu
from jax.experimental.pallas import tpu_sc as plsc

N = 10000
NPAD = 10240
E = 160000
EPAD = 163840
ECH = EPAD // 128
F_IN = 1433
F_PAD = 1536
HID = 256
C = 7
CP = 16
K = 3

_MESH = dict(core_axis_name="c", subcore_axis_name="s")


def _zero_acc(acc, zbuf, s):
    def zrow(i, _):
        for f in range(8):
            zbuf[i, pl.ds(f * 16, 16)] = jnp.zeros((16,), jnp.float32)
        return 0
    lax.fori_loop(0, 8, zrow, 0)

    def zcp(b, _):
        pltpu.sync_copy(zbuf, acc.at[pl.ds(s * 640 + b * 8, 8)])
        return 0
    lax.fori_loop(0, 80, zcp, 0)


def _deg_kernel(dst_hbm, out_hbm, acc, dstv, ones01, zbuf, sem):
    c = lax.axis_index("c")
    s = lax.axis_index("s")

    _zero_acc(acc, zbuf, s)

    def orow(i, _):
        ones01[i, pl.ds(0, 16)] = jnp.where(
            lax.iota(jnp.int32, 16) == 0, 1.0, 0.0)
        for f in range(1, 8):
            ones01[i, pl.ds(f * 16, 16)] = jnp.zeros((16,), jnp.float32)
        return 0
    lax.fori_loop(0, 128, orow, 0)
    plsc.subcore_barrier()

    pltpu.sync_copy(dst_hbm.at[pl.ds(c * 640 + s * 40, 40)], dstv)

    def ebody(j, _):
        pltpu.async_copy(ones01, acc.at[dstv.at[j]], sem, add=True)
        return 0
    lax.fori_loop(0, 40, ebody, 0)

    def edrain(j, _):
        pltpu.make_async_copy(ones01, acc.at[dstv.at[j]], sem).wait()
        return 0
    lax.fori_loop(0, 40, edrain, 0)
    plsc.subcore_barrier()

    r0 = s * 640
    pltpu.sync_copy(acc.at[pl.ds(r0, 640)], out_hbm.at[c].at[pl.ds(r0, 640)])


def _degree(dst2d):
    k = pl.kernel(
        _deg_kernel,
        out_type=jax.ShapeDtypeStruct((2, NPAD, 128), jnp.float32),
        mesh=plsc.VectorSubcoreMesh(**_MESH),
        scratch_types=[
            pltpu.VMEM_SHARED((NPAD, 128), jnp.float32),
            pltpu.VMEM((40, 128), jnp.int32),
            pltpu.VMEM((128, 128), jnp.float32),
            pltpu.VMEM((8, 128), jnp.float32),
            pltpu.SemaphoreType.DMA,
        ],
    )
    return k(dst2d)


def _scales_body(deg_ref, dis_ref, dis2_ref):
    d = deg_ref[0, :, 0:1] + deg_ref[1, :, 0:1]
    pos = d > 0.0
    dsafe = jnp.maximum(d, 1e-12)
    ones = jnp.ones((1, 16), jnp.float32)
    dis_ref[...] = jnp.where(pos, lax.rsqrt(dsafe), 0.0) * ones
    dis2_ref[...] = jnp.where(pos, 1.0 / dsafe, 0.0) * ones


def _scales(deg16):
    return pl.pallas_call(
        _scales_body,
        grid=(10,),
        in_specs=[pl.BlockSpec((2, 1024, 128), lambda i: (0, i, 0))],
        out_specs=[
            pl.BlockSpec((1024, 16), lambda i: (i, 0)),
            pl.BlockSpec((1024, 16), lambda i: (i, 0)),
        ],
        out_shape=[
            jax.ShapeDtypeStruct((NPAD, 16), jnp.float32),
            jax.ShapeDtypeStruct((NPAD, 16), jnp.float32),
        ],
    )(deg16)


def _m1_body(x_ref, w_ref, dis_ref, y_ref, g3_ref):
    j = pl.program_id(1)
    acc = jnp.dot(x_ref[...], w_ref[...], preferred_element_type=jnp.float32)
    ys = jnp.where(j >= 2, acc * dis_ref[:, 0:1], acc)
    y_ref[0, 0] = ys
    g3_ref[0] = ys


def _matmul1(x_pad, w1r, dis16):
    return pl.pallas_call(
        _m1_body,
        grid=(20, 8),
        in_specs=[
            pl.BlockSpec((512, F_PAD), lambda i, j: (i, 0)),
            pl.BlockSpec((F_PAD, 128), lambda i, j: (0, j)),
            pl.BlockSpec((512, 16), lambda i, j: (i, 0)),
        ],
        out_specs=[
            pl.BlockSpec((1, 1, 512, 128), lambda i, j: (j // 2, j % 2, i, 0)),
            pl.BlockSpec((1, 512, 128), lambda i, j: (j % 2, i, 0)),
        ],
        out_shape=[
            jax.ShapeDtypeStruct((K + 1, 2, NPAD, 128), jnp.float32),
            jax.ShapeDtypeStruct((2, NPAD, 128), jnp.float32),
        ],
    )(x_pad, w1r, dis16)


def _edge_pipeline(table, srcv, dstv, acc, rows0, rows1, sem, nchunks):
    pltpu.async_copy(table.at[srcv.at[0]], rows0, sem)

    def ebody(jj, _):
        j0 = 2 * jj
        j1 = j0 + 1
        pltpu.make_async_copy(table.at[srcv.at[j0]], rows0, sem).wait()
        pltpu.async_copy(table.at[srcv.at[j1]], rows1, sem)
        pltpu.sync_copy(rows0, acc.at[dstv.at[j0]], add=True)
        pltpu.make_async_copy(table.at[srcv.at[j1]], rows1, sem).wait()

        @pl.when(jj < nchunks // 2 - 1)
        def _():
            pltpu.async_copy(table.at[srcv.at[j0 + 2]], rows0, sem)

        pltpu.sync_copy(rows1, acc.at[dstv.at[j1]], add=True)
        return 0
    lax.fori_loop(0, nchunks // 2, ebody, 0)


def _s1_kernel(g_hbm, src_hbm, dst_hbm, out_hbm, acc, srcv, dstv, rows0,
               rows1, zbuf, sem):
    c = lax.axis_index("c")
    s = lax.axis_index("s")

    _zero_acc(acc, zbuf, s)
    plsc.subcore_barrier()

    for h in range(2):
        pltpu.sync_copy(src_hbm.at[pl.ds(s * 80 + h * 40, 40)], srcv)
        pltpu.sync_copy(dst_hbm.at[pl.ds(s * 80 + h * 40, 40)], dstv)
        _edge_pipeline(g_hbm.at[c], srcv, dstv, acc, rows0, rows1, sem, 40)
    plsc.subcore_barrier()

    r0 = s * 640
    pltpu.sync_copy(acc.at[pl.ds(r0, 640)], out_hbm.at[c].at[pl.ds(r0, 640)])


def _agg1(g, src2d, dst2d):
    k = pl.kernel(
        _s1_kernel,
        out_type=jax.ShapeDtypeStruct((2, NPAD, 128), jnp.float32),
        mesh=plsc.VectorSubcoreMesh(**_MESH),
        scratch_types=[
            pltpu.VMEM_SHARED((NPAD, 128), jnp.float32),
            pltpu.VMEM((40, 128), jnp.int32),
            pltpu.VMEM((40, 128), jnp.int32),
            pltpu.VMEM((128, 128), jnp.float32),
            pltpu.VMEM((128, 128), jnp.float32),
            pltpu.VMEM((8, 128), jnp.float32),
            pltpu.SemaphoreType.DMA,
        ],
    )
    return k(g, src2d, dst2d)


def _c1_body(kk, yh_ref, s_ref, d2_ref, out_ref):
    del kk
    out_ref[0] = yh_ref[0, 0] + d2_ref[:, 0:1] * s_ref[0]


def _combine1(kk, y4, s, dis2_16):
    return pl.pallas_call(
        functools.partial(_c1_body, kk),
        grid=(2, 20),
        in_specs=[
            pl.BlockSpec((1, 1, 512, 128), lambda c, i: (kk, c, i, 0)),
            pl.BlockSpec((1, 512, 128), lambda c, i: (c, i, 0)),
            pl.BlockSpec((512, 16), lambda c, i: (i, 0)),
        ],
        out_specs=pl.BlockSpec((1, 512, 128), lambda c, i: (c, i, 0)),
        out_shape=jax.ShapeDtypeStruct((2, NPAD, 128), jnp.float32),
    )(y4, s, dis2_16)


def _m2_body(s_ref, y4_ref, dis_ref, b1_ref, w_ref, y2_ref, g2_ref):
    c = pl.program_id(1)
    h = jnp.maximum(
        y4_ref[0, 0] + dis_ref[:, 0:1] * s_ref[0] + b1_ref[0], 0.0)

    @pl.when(c == 0)
    def _():
        y2_ref[...] = jnp.zeros_like(y2_ref)

    y2_ref[...] += jnp.dot(h, w_ref[0], preferred_element_type=jnp.float32)

    @pl.when(c == 1)
    def _():
        col = lax.broadcasted_iota(jnp.int32, (512, (K + 1) * CP), 1)
        y2s = jnp.where(col < CP, y2_ref[...], y2_ref[...] * dis_ref[:, 0:1])
        y2_ref[...] = y2s
        g2_ref[...] = jnp.concatenate(
            [y2s[:, K * CP:(K + 1) * CP],
             jnp.zeros((512, 128 - CP), jnp.float32)], axis=1)


def _matmul2(s, y4, dis16, b1r, w2p):
    return pl.pallas_call(
        _m2_body,
        grid=(20, 2),
        in_specs=[
            pl.BlockSpec((1, 512, 128), lambda i, c: (c, i, 0)),
            pl.BlockSpec((1, 1, 512, 128), lambda i, c: (0, c, i, 0)),
            pl.BlockSpec((512, 16), lambda i, c: (i, 0)),
            pl.BlockSpec((1, 1, 128), lambda i, c: (c, 0, 0)),
            pl.BlockSpec((1, 128, (K + 1) * CP), lambda i, c: (c, 0, 0)),
        ],
        out_specs=[
            pl.BlockSpec((512, (K + 1) * CP), lambda i, c: (i, 0)),
            pl.BlockSpec((512, 128), lambda i, c: (i, 0)),
        ],
        out_shape=[
            jax.ShapeDtypeStruct((NPAD, (K + 1) * CP), jnp.float32),
            jax.ShapeDtypeStruct((NPAD, 128), jnp.float32),
        ],
    )(s, y4, dis16, b1r, w2p)


def _s2_kernel(g_hbm, src_hbm, dst_hbm, out_hbm, acc, srcv, dstv, rows0,
               rows1, zbuf, sem):
    c = lax.axis_index("c")
    s = lax.axis_index("s")

    _zero_acc(acc, zbuf, s)
    plsc.subcore_barrier()

    pltpu.sync_copy(src_hbm.at[pl.ds(c * 640 + s * 40, 40)], srcv)
    pltpu.sync_copy(dst_hbm.at[pl.ds(c * 640 + s * 40, 40)], dstv)
    _edge_pipeline(g_hbm, srcv, dstv, acc, rows0, rows1, sem, 40)
    plsc.subcore_barrier()

    r0 = s * 640
    pltpu.sync_copy(acc.at[pl.ds(r0, 640)], out_hbm.at[c].at[pl.ds(r0, 640)])


def _agg2(g, src2d, dst2d):
    k = pl.kernel(
        _s2_kernel,
        out_type=jax.ShapeDtypeStruct((2, NPAD, 128), jnp.float32),
        mesh=plsc.VectorSubcoreMesh(**_MESH),
        scratch_types=[
            pltpu.VMEM_SHARED((NPAD, 128), jnp.float32),
            pltpu.VMEM((40, 128), jnp.int32),
            pltpu.VMEM((40, 128), jnp.int32),
            pltpu.VMEM((128, 128), jnp.float32),
            pltpu.VMEM((128, 128), jnp.float32),
            pltpu.VMEM((8, 128), jnp.float32),
            pltpu.SemaphoreType.DMA,
        ],
    )
    return k(g, src2d, dst2d)


def _c2_body(yh_ref, s_ref, a2_ref, out_ref):
    sv = s_ref[0, :, 0:CP] + s_ref[1, :, 0:CP]
    v = yh_ref[0] + a2_ref[:, 0:1] * sv
    out_ref[...] = jnp.concatenate(
        [v, jnp.zeros((1024, 128 - CP), jnp.float32)], axis=1)


def _combine2(kk, y2k, s, a2_16):
    return pl.pallas_call(
        _c2_body,
        grid=(10,),
        in_specs=[
            pl.BlockSpec((1, 1024, CP), lambda i: (kk, i, 0)),
            pl.BlockSpec((2, 1024, 128), lambda i: (0, i, 0)),
            pl.BlockSpec((1024, 16), lambda i: (i, 0)),
        ],
        out_specs=pl.BlockSpec((1024, 128), lambda i: (i, 0)),
        out_shape=jax.ShapeDtypeStruct((NPAD, 128), jnp.float32),
    )(y2k, s, a2_16)


@jax.jit
def _run(x, edge_index, W1, b1, W2, b2):
    src = edge_index[0]
    dst = edge_index[1]
    src2d = jnp.concatenate(
        [src, jnp.zeros((EPAD - E,), jnp.int32)]).reshape(ECH, 128)
    dst2d = jnp.concatenate(
        [dst, jnp.full((EPAD - E,), N, jnp.int32)]).reshape(ECH, 128)

    x_pad = jnp.zeros((NPAD, F_PAD), jnp.float32).at[:N, :F_IN].set(x)
    w1r = jnp.zeros((F_PAD, (K + 1) * HID), jnp.float32).at[:F_IN].set(
        jnp.transpose(W1, (1, 0, 2)).reshape(F_IN, (K + 1) * HID))
    w2p = jnp.zeros((HID, K + 1, CP), jnp.float32).at[:, :, :C].set(
        jnp.transpose(W2, (1, 0, 2))).reshape(2, 128, (K + 1) * CP)
    b1r = b1.reshape(2, 1, 128)

    deg16 = _degree(dst2d)
    dis16, dis2_16 = _scales(deg16)

    y4, g = _matmul1(x_pad, w1r, dis16)
    g = _combine1(2, y4, _agg1(g, src2d, dst2d), dis2_16)
    g = _combine1(1, y4, _agg1(g, src2d, dst2d), dis2_16)
    sl = _agg1(g, src2d, dst2d)

    y2, g2 = _matmul2(sl, y4, dis16, b1r, w2p)
    y2k = jnp.transpose(y2.reshape(NPAD, K + 1, CP), (1, 0, 2))
    g2 = _combine2(2, y2k, _agg2(g2, src2d, dst2d), dis2_16)
    g2 = _combine2(1, y2k, _agg2(g2, src2d, dst2d), dis2_16)
    out = _combine2(0, y2k, _agg2(g2, src2d, dst2d), dis16)

    return out[:N, :C] + b2


def kernel(x, edge_index, W1, b1, W2, b2):
    return _run(x, edge_index, W1, b1, W2, b2)
